# baseline (speedup 1.0000x reference)
import jax
import jax.numpy as jnp
from jax.experimental import pallas as pl
from jax.experimental.pallas import tpu as pltpu

_BLOCK = 512


def _copy_kernel(flat_ref, rs_ref, flat_out, rs_out):
    flat_out[...] = flat_ref[...]

    @pl.when(pl.program_id(0) == 0)
    def _():
        for i in range(rs_ref.shape[0]):
            rs_out[i] = rs_ref[i]


def kernel(flat, row_splits):
    n_rows, n_feat = flat.shape
    grid = (n_rows // _BLOCK,)
    return pl.pallas_call(
        _copy_kernel,
        grid=grid,
        out_shape=(
            jax.ShapeDtypeStruct(flat.shape, flat.dtype),
            jax.ShapeDtypeStruct(row_splits.shape, row_splits.dtype),
        ),
        in_specs=[
            pl.BlockSpec((_BLOCK, n_feat), lambda i: (i, 0)),
            pl.BlockSpec(memory_space=pltpu.MemorySpace.SMEM),
        ],
        out_specs=(
            pl.BlockSpec((_BLOCK, n_feat), lambda i: (i, 0)),
            pl.BlockSpec(memory_space=pltpu.MemorySpace.SMEM),
        ),
    )(flat, row_splits)


# speedup vs baseline: 1.5686x; 1.5686x over previous
import jax
import jax.numpy as jnp
from jax.experimental import pallas as pl
from jax.experimental.pallas import tpu as pltpu

_BLOCK = 4096


def _copy_kernel(flat_ref, rs_ref, flat_out, rs_out):
    flat_out[...] = flat_ref[...]

    @pl.when(pl.program_id(0) == 0)
    def _():
        for i in range(rs_ref.shape[0]):
            rs_out[i] = rs_ref[i]


def kernel(flat, row_splits):
    n_rows, n_feat = flat.shape
    grid = (n_rows // _BLOCK,)
    return pl.pallas_call(
        _copy_kernel,
        grid=grid,
        out_shape=(
            jax.ShapeDtypeStruct(flat.shape, flat.dtype),
            jax.ShapeDtypeStruct(row_splits.shape, row_splits.dtype),
        ),
        in_specs=[
            pl.BlockSpec((_BLOCK, n_feat), lambda i: (i, 0)),
            pl.BlockSpec(memory_space=pltpu.MemorySpace.SMEM),
        ],
        out_specs=(
            pl.BlockSpec((_BLOCK, n_feat), lambda i: (i, 0)),
            pl.BlockSpec(memory_space=pltpu.MemorySpace.SMEM),
        ),
    )(flat, row_splits)
